# baseline (speedup 1.0000x reference)
import functools

import jax
import jax.numpy as jnp
from jax import lax
from jax.experimental import pallas as pl
from jax.experimental.pallas import tpu as pltpu
from jax.experimental.pallas import tpu_sc as plsc

N = 10000
D = 128
G = 64
NCLS = 32

NP = 10240
NWORK = 32
CH = 128
NCHUNK = 88
EW = CH * NCHUNK
EPAD = EW * NWORK
SL = NP // 16

_f32 = jnp.float32
_i32 = jnp.int32


_BLK = 512


def _tc1_body(x_ref, w_ref, a2_ref, h_ref, asad_ref):
    i = pl.program_id(0)
    h = jnp.dot(x_ref[...], w_ref[...], preferred_element_type=_f32)
    h_ref[...] = h
    asad = jnp.dot(h, a2_ref[...], preferred_element_type=_f32).T
    ids = i * _BLK + lax.broadcasted_iota(_i32, (2, _BLK), 1)
    asad_ref[...] = jnp.where(ids < N, asad, -1e30)


def _tc_layer1(x_pad, W, a2):
    return pl.pallas_call(
        _tc1_body,
        grid=(NP // _BLK,),
        in_specs=[
            pl.BlockSpec((_BLK, D), lambda i: (i, 0)),
            pl.BlockSpec((D, D), lambda i: (0, 0)),
            pl.BlockSpec((D, 2), lambda i: (0, 0)),
        ],
        out_specs=[
            pl.BlockSpec((_BLK, D), lambda i: (i, 0)),
            pl.BlockSpec((2, _BLK), lambda i: (0, i)),
        ],
        out_shape=[
            jax.ShapeDtypeStruct((NP, D), _f32),
            jax.ShapeDtypeStruct((2, NP), _f32),
        ],
    )(x_pad, W, a2)


def _tc2_body(g0_ref, g1_ref, den_ref, b_ref, w_ref, a2_ref, h_ref, asad_ref):
    i = pl.program_id(0)
    den = den_ref[0] + den_ref[1]
    inv = 1.0 / (den + 1e-16)
    hin = jax.nn.relu((g0_ref[...] + g1_ref[...]) * inv[:, None] + b_ref[...])
    h = jnp.dot(hin, w_ref[...], preferred_element_type=_f32)
    h_ref[...] = h
    asad = jnp.dot(h, a2_ref[...], preferred_element_type=_f32).T
    ids = i * _BLK + lax.broadcasted_iota(_i32, (2, _BLK), 1)
    asad_ref[...] = jnp.where(ids < N, asad, -1e30)


def _tc_layer23(agg0, agg1, den, b_prev, W, a2):
    return pl.pallas_call(
        _tc2_body,
        grid=(NP // _BLK,),
        in_specs=[
            pl.BlockSpec((_BLK, D), lambda i: (i, 0)),
            pl.BlockSpec((_BLK, D), lambda i: (i, 0)),
            pl.BlockSpec((2, _BLK), lambda i: (0, i)),
            pl.BlockSpec((1, D), lambda i: (0, 0)),
            pl.BlockSpec((D, D), lambda i: (0, 0)),
            pl.BlockSpec((D, 2), lambda i: (0, 0)),
        ],
        out_specs=[
            pl.BlockSpec((_BLK, D), lambda i: (i, 0)),
            pl.BlockSpec((2, _BLK), lambda i: (0, i)),
        ],
        out_shape=[
            jax.ShapeDtypeStruct((NP, D), _f32),
            jax.ShapeDtypeStruct((2, NP), _f32),
        ],
    )(agg0, agg1, den, b_prev, W, a2)


def _tcf_body(g0_ref, g1_ref, den_ref, b_ref, batch_ref, l1w_ref, l1b_ref,
              l2w_ref, l2b_ref, out_ref):
    den = den_ref[0] + den_ref[1]
    inv = 1.0 / (den + 1e-16)
    h3 = jax.nn.relu((g0_ref[...] + g1_ref[...]) * inv[:, None]
                     + b_ref[...])
    gid = lax.broadcasted_iota(_i32, (G, NP), 0)
    onehot = (batch_ref[...] == gid).astype(_f32)
    cnt = jnp.sum(onehot, axis=1, keepdims=True)
    pooled = jnp.dot(onehot, h3, preferred_element_type=_f32)
    pooled = pooled / jnp.maximum(cnt, 1.0)
    s1 = jax.nn.relu(jnp.dot(pooled, l1w_ref[...],
                             preferred_element_type=_f32) + l1b_ref[...])
    lg = jnp.dot(s1, l2w_ref[...], preferred_element_type=_f32) + l2b_ref[...]
    m = jnp.max(lg, axis=0, keepdims=True)
    z = lg - m
    out_ref[...] = z - jnp.log(jnp.sum(jnp.exp(z), axis=0, keepdims=True))


def _tc_final(agg0, agg1, den, b3, batch2d, l1w, l1b, l2w, l2b):
    return pl.pallas_call(
        _tcf_body,
        out_shape=jax.ShapeDtypeStruct((G, NCLS), _f32),
    )(agg0, agg1, den, b3, batch2d, l1w, l1b, l2w, l2b)


_MESH = plsc.VectorSubcoreMesh(core_axis_name="c", subcore_axis_name="s")
_SC_PARAMS = pltpu.CompilerParams(needs_layout_passes=False)


def _lrelu(e):
    return jnp.where(e >= 0.0, e, 0.2 * e)


BRA = 8
NCHUNKA = NCHUNK // BRA


def _sc_denom_kernel(src_hbm, dst_hbm, asad_hbm, den_hbm, p_hbm,
                     as_v, ad_v, s0, s1, d0, d1, p0, p1, den_l, acc_v, tmp_v,
                     shr_den, se0, se1, pe0, pe1):
    cid = lax.axis_index("c")
    sid = lax.axis_index("s")
    row_base = (cid * 16 + sid) * NCHUNK

    pltpu.sync_copy(asad_hbm.at[0], as_v)
    pltpu.sync_copy(asad_hbm.at[1], ad_v)

    def zbody(i, _):
        den_l[pl.ds(i * 16, 16)] = jnp.zeros((16,), _f32)
        return 0
    lax.fori_loop(0, NP // 16, zbody, 0)

    sb, db, pb = (s0, s1), (d0, d1), (p0, p1)
    ssem, psem = (se0, se1), (pe0, pe1)
    stg = [None] * NCHUNKA
    pw = [None] * NCHUNKA

    def stage(ci):
        r0 = row_base + ci * BRA
        b = ci & 1
        return (pltpu.async_copy(src_hbm.at[pl.ds(r0, BRA)], sb[b], ssem[b]),
                pltpu.async_copy(dst_hbm.at[pl.ds(r0, BRA)], db[b], ssem[b]))

    stg[0] = stage(0)
    for ci in range(NCHUNKA):
        b = ci & 1
        if ci + 1 < NCHUNKA:
            stg[ci + 1] = stage(ci + 1)
        stg[ci][0].wait()
        stg[ci][1].wait()
        if ci >= 2:
            pw[ci - 2].wait()

        def rowbody(r, _):
            def lane(li, _):
                sl = pl.ds(li * 16, 16)
                s = sb[b][r, sl]
                d = db[b][r, sl]
                e = _lrelu(plsc.load_gather(as_v, [s])
                           + plsc.load_gather(ad_v, [d]))
                p = jnp.exp(e)
                pb[b][r, sl] = p
                plsc.addupdate_scatter(den_l, [d], p)
                return 0
            lax.fori_loop(0, CH // 16, lane, 0)
            return 0
        lax.fori_loop(0, BRA, rowbody, 0)
        pw[ci] = pltpu.async_copy(
            pb[b], p_hbm.at[pl.ds(row_base + ci * BRA, BRA)], psem[b])
    pw[NCHUNKA - 2].wait()
    pw[NCHUNKA - 1].wait()

    pltpu.sync_copy(den_l, shr_den.at[sid])
    plsc.subcore_barrier()
    pltpu.sync_copy(shr_den.at[0, pl.ds(sid * SL, SL)], acc_v)

    def rbody(t, _):
        pltpu.sync_copy(shr_den.at[t, pl.ds(sid * SL, SL)], tmp_v)

        def abody(i, _):
            sl = pl.ds(i * 16, 16)
            acc_v[sl] = acc_v[sl] + tmp_v[sl]
            return 0
        lax.fori_loop(0, SL // 16, abody, 0)
        return 0
    lax.fori_loop(1, 16, rbody, 0)
    pltpu.sync_copy(acc_v, den_hbm.at[cid, pl.ds(sid * SL, SL)])


def _sc_denom(src2, dst2, asad):
    k = pl.kernel(
        _sc_denom_kernel,
        out_type=(jax.ShapeDtypeStruct((2, NP), _f32),
                  jax.ShapeDtypeStruct((EPAD // CH, CH), _f32)),
        mesh=_MESH,
        compiler_params=_SC_PARAMS,
        scratch_types=[
            pltpu.VMEM((NP,), _f32),
            pltpu.VMEM((NP,), _f32),
            pltpu.VMEM((BRA, CH), _i32),
            pltpu.VMEM((BRA, CH), _i32),
            pltpu.VMEM((BRA, CH), _i32),
            pltpu.VMEM((BRA, CH), _i32),
            pltpu.VMEM((BRA, CH), _f32),
            pltpu.VMEM((BRA, CH), _f32),
            pltpu.VMEM((NP,), _f32),
            pltpu.VMEM((SL,), _f32),
            pltpu.VMEM((SL,), _f32),
            pltpu.VMEM_SHARED((16, NP), _f32),
            pltpu.SemaphoreType.DMA,
            pltpu.SemaphoreType.DMA,
            pltpu.SemaphoreType.DMA,
            pltpu.SemaphoreType.DMA,
        ],
    )
    return k(src2, dst2, asad)


NBLK = 11
BCH = NCHUNK // NBLK


def _sc_agg_kernel(src_hbm, dst_hbm, p_hbm, h_hbm, agg0_hbm,
                   agg1_hbm, si0, si1, di0, di1, pv0, pv1, rows0, rows1,
                   out_shr, gs0, gs1, ss0, ss1, ts0, ts1):
    cid = lax.axis_index("c")
    sid = lax.axis_index("s")
    row_base = (cid * 16 + sid) * NCHUNK

    def zr(i, _):
        for j in range(8):
            rows0[i, pl.ds(j * 16, 16)] = jnp.zeros((16,), _f32)
        return 0
    lax.fori_loop(0, CH, zr, 0)
    for kk in range(SL // CH):
        pltpu.sync_copy(rows0, out_shr.at[pl.ds(sid * SL + kk * CH, CH)])

    plsc.subcore_barrier()

    rows = (rows0, rows1)
    sb, db, pb = (si0, si1), (di0, di1), (pv0, pv1)
    gsem = (gs0, gs1)
    ssem = (ss0, ss1)
    tsem = (ts0, ts1)

    def stage(g, b):
        r0 = row_base + g * BCH
        pltpu.async_copy(src_hbm.at[pl.ds(r0, BCH)], sb[b], tsem[b])
        pltpu.async_copy(dst_hbm.at[pl.ds(r0, BCH)], db[b], tsem[b])
        pltpu.async_copy(p_hbm.at[pl.ds(r0, BCH)], pb[b], tsem[b])

    def stage_wait(b):
        pltpu.make_async_copy(src_hbm.at[pl.ds(0, BCH)], sb[b], tsem[b]).wait()
        pltpu.make_async_copy(dst_hbm.at[pl.ds(0, BCH)], db[b], tsem[b]).wait()
        pltpu.make_async_copy(p_hbm.at[pl.ds(0, BCH)], pb[b], tsem[b]).wait()

    def run_block(g, bg, last):
        if not last:
            stage(g + 1, 1 - bg)
        stage_wait(bg)
        gd = [None] * BCH
        sd = [None] * BCH
        gd[0] = pltpu.async_copy(h_hbm.at[sb[bg].at[0]], rows[0], gsem[0])
        for k in range(BCH):
            b = k & 1
            if k + 1 < BCH:
                if k >= 1:
                    sd[k - 1].wait()
                gd[k + 1] = pltpu.async_copy(
                    h_hbm.at[sb[bg].at[k + 1]], rows[(k + 1) & 1],
                    gsem[(k + 1) & 1])
            gd[k].wait()
            kvec = jnp.full((16,), k, _i32)

            def sbody(i, _):
                i0 = i * 4
                wv = [plsc.load_gather(
                    pb[bg], [kvec, jnp.full((16,), u, _i32) + i0])
                    for u in range(4)]
                for u in range(4):
                    for j in range(8):
                        sl = pl.ds(j * 16, 16)
                        rows[b][i0 + u, sl] = rows[b][i0 + u, sl] * wv[u]
                return 0
            lax.fori_loop(0, CH // 4, sbody, 0)

            sd[k] = pltpu.async_copy(rows[b], out_shr.at[db[bg].at[k]],
                                     ssem[b], add=True)
        sd[BCH - 2].wait()
        sd[BCH - 1].wait()

    stage(0, 0)

    def pair(q, _):
        run_block(2 * q, 0, False)
        run_block(2 * q + 1, 1, False)
        return 0
    lax.fori_loop(0, NBLK // 2, pair, 0)
    run_block(NBLK - 1, 0, True)

    plsc.subcore_barrier()
    for kk in range(SL // CH):
        r0 = sid * SL + kk * CH

        @pl.when(cid == 0)
        def _():
            pltpu.sync_copy(out_shr.at[pl.ds(r0, CH)],
                            agg0_hbm.at[pl.ds(r0, CH)])

        @pl.when(cid == 1)
        def _():
            pltpu.sync_copy(out_shr.at[pl.ds(r0, CH)],
                            agg1_hbm.at[pl.ds(r0, CH)])


def _sc_agg(src2, dst2, p2, h):
    k = pl.kernel(
        _sc_agg_kernel,
        out_type=(jax.ShapeDtypeStruct((NP, D), _f32),
                  jax.ShapeDtypeStruct((NP, D), _f32)),
        mesh=_MESH,
        compiler_params=_SC_PARAMS,
        scratch_types=[
            pltpu.VMEM((BCH, CH), _i32),
            pltpu.VMEM((BCH, CH), _i32),
            pltpu.VMEM((BCH, CH), _i32),
            pltpu.VMEM((BCH, CH), _i32),
            pltpu.VMEM((BCH, CH), _f32),
            pltpu.VMEM((BCH, CH), _f32),
            pltpu.VMEM((CH, D), _f32),
            pltpu.VMEM((CH, D), _f32),
            pltpu.VMEM_SHARED((NP, D), _f32),
            pltpu.SemaphoreType.DMA,
            pltpu.SemaphoreType.DMA,
            pltpu.SemaphoreType.DMA,
            pltpu.SemaphoreType.DMA,
            pltpu.SemaphoreType.DMA,
            pltpu.SemaphoreType.DMA,
        ],
    )
    return k(src2, dst2, p2, h)


def kernel(x, edge_index, batch, W1, a_src1, a_dst1, b1, W2, a_src2, a_dst2,
           b2, W3, a_src3, a_dst3, b3, lin1_W, lin1_b, lin2_W, lin2_b):
    loop = jnp.arange(N, dtype=edge_index.dtype)
    src = jnp.concatenate([edge_index[0], loop])
    dst = jnp.concatenate([edge_index[1], loop])
    pad = N + jnp.arange(EPAD - src.shape[0], dtype=_i32) % (NP - N)
    src = jnp.concatenate([src.astype(_i32), pad])
    dst = jnp.concatenate([dst.astype(_i32), pad])

    x_pad = jnp.pad(x, ((0, NP - N), (0, 0)))
    batch2d = jnp.pad(batch.astype(_i32), (0, NP - N),
                      constant_values=G).reshape(1, NP)

    a21 = jnp.stack([a_src1, a_dst1], axis=1)
    a22 = jnp.stack([a_src2, a_dst2], axis=1)
    a23 = jnp.stack([a_src3, a_dst3], axis=1)

    src2 = src.reshape(EPAD // CH, CH)
    dst2 = dst.reshape(EPAD // CH, CH)

    h1, asad1 = _tc_layer1(x_pad, W1, a21)
    den1, p1 = _sc_denom(src2, dst2, asad1)
    agg1a, agg1b = _sc_agg(src2, dst2, p1, h1)

    h2, asad2 = _tc_layer23(agg1a, agg1b, den1, b1.reshape(1, D), W2, a22)
    den2, p2 = _sc_denom(src2, dst2, asad2)
    agg2a, agg2b = _sc_agg(src2, dst2, p2, h2)

    h3, asad3 = _tc_layer23(agg2a, agg2b, den2, b2.reshape(1, D), W3, a23)
    den3, p3 = _sc_denom(src2, dst2, asad3)
    agg3a, agg3b = _sc_agg(src2, dst2, p3, h3)

    return _tc_final(agg3a, agg3b, den3, b3.reshape(1, D), batch2d, lin1_W,
                     lin1_b.reshape(1, -1), lin2_W, lin2_b.reshape(1, -1))


# speedup vs baseline: 1.0114x; 1.0114x over previous
import functools

import jax
import jax.numpy as jnp
from jax import lax
from jax.experimental import pallas as pl
from jax.experimental.pallas import tpu as pltpu
from jax.experimental.pallas import tpu_sc as plsc

N = 10000
D = 128
G = 64
NCLS = 32

NP = 10240
NWORK = 32
CH = 128
NCHUNK = 88
EW = CH * NCHUNK
EPAD = EW * NWORK
SL = NP // 16

_f32 = jnp.float32
_i32 = jnp.int32


_BLK = 512


def _tc1_body(x_ref, w_ref, a2_ref, h_ref, asad_ref):
    i = pl.program_id(0)
    h = jnp.dot(x_ref[...], w_ref[...], preferred_element_type=_f32)
    h_ref[...] = h
    asad = jnp.dot(h, a2_ref[...], preferred_element_type=_f32).T
    ids = i * _BLK + lax.broadcasted_iota(_i32, (2, _BLK), 1)
    asad_ref[...] = jnp.where(ids < N, asad, -1e30)


def _tc_layer1(x_pad, W, a2):
    return pl.pallas_call(
        _tc1_body,
        grid=(NP // _BLK,),
        in_specs=[
            pl.BlockSpec((_BLK, D), lambda i: (i, 0)),
            pl.BlockSpec((D, D), lambda i: (0, 0)),
            pl.BlockSpec((D, 2), lambda i: (0, 0)),
        ],
        out_specs=[
            pl.BlockSpec((_BLK, D), lambda i: (i, 0)),
            pl.BlockSpec((2, _BLK), lambda i: (0, i)),
        ],
        out_shape=[
            jax.ShapeDtypeStruct((NP, D), _f32),
            jax.ShapeDtypeStruct((2, NP), _f32),
        ],
    )(x_pad, W, a2)


def _tc2_body(g0_ref, g1_ref, den_ref, b_ref, w_ref, a2_ref, h_ref, asad_ref):
    i = pl.program_id(0)
    den = den_ref[0] + den_ref[1]
    inv = 1.0 / (den + 1e-16)
    hin = jax.nn.relu((g0_ref[...] + g1_ref[...]) * inv[:, None] + b_ref[...])
    h = jnp.dot(hin, w_ref[...], preferred_element_type=_f32)
    h_ref[...] = h
    asad = jnp.dot(h, a2_ref[...], preferred_element_type=_f32).T
    ids = i * _BLK + lax.broadcasted_iota(_i32, (2, _BLK), 1)
    asad_ref[...] = jnp.where(ids < N, asad, -1e30)


def _tc_layer23(agg0, agg1, den, b_prev, W, a2):
    return pl.pallas_call(
        _tc2_body,
        grid=(NP // _BLK,),
        in_specs=[
            pl.BlockSpec((_BLK, D), lambda i: (i, 0)),
            pl.BlockSpec((_BLK, D), lambda i: (i, 0)),
            pl.BlockSpec((2, _BLK), lambda i: (0, i)),
            pl.BlockSpec((1, D), lambda i: (0, 0)),
            pl.BlockSpec((D, D), lambda i: (0, 0)),
            pl.BlockSpec((D, 2), lambda i: (0, 0)),
        ],
        out_specs=[
            pl.BlockSpec((_BLK, D), lambda i: (i, 0)),
            pl.BlockSpec((2, _BLK), lambda i: (0, i)),
        ],
        out_shape=[
            jax.ShapeDtypeStruct((NP, D), _f32),
            jax.ShapeDtypeStruct((2, NP), _f32),
        ],
    )(agg0, agg1, den, b_prev, W, a2)


def _tcf_body(g0_ref, g1_ref, den_ref, b_ref, batch_ref, l1w_ref, l1b_ref,
              l2w_ref, l2b_ref, out_ref):
    den = den_ref[0] + den_ref[1]
    inv = 1.0 / (den + 1e-16)
    h3 = jax.nn.relu((g0_ref[...] + g1_ref[...]) * inv[:, None]
                     + b_ref[...])
    gid = lax.broadcasted_iota(_i32, (G, NP), 0)
    onehot = (batch_ref[...] == gid).astype(_f32)
    cnt = jnp.sum(onehot, axis=1, keepdims=True)
    pooled = jnp.dot(onehot, h3, preferred_element_type=_f32)
    pooled = pooled / jnp.maximum(cnt, 1.0)
    s1 = jax.nn.relu(jnp.dot(pooled, l1w_ref[...],
                             preferred_element_type=_f32) + l1b_ref[...])
    lg = jnp.dot(s1, l2w_ref[...], preferred_element_type=_f32) + l2b_ref[...]
    m = jnp.max(lg, axis=0, keepdims=True)
    z = lg - m
    out_ref[...] = z - jnp.log(jnp.sum(jnp.exp(z), axis=0, keepdims=True))


def _tc_final(agg0, agg1, den, b3, batch2d, l1w, l1b, l2w, l2b):
    return pl.pallas_call(
        _tcf_body,
        out_shape=jax.ShapeDtypeStruct((G, NCLS), _f32),
    )(agg0, agg1, den, b3, batch2d, l1w, l1b, l2w, l2b)


_MESH = plsc.VectorSubcoreMesh(core_axis_name="c", subcore_axis_name="s")
_SC_PARAMS = pltpu.CompilerParams(needs_layout_passes=False)


def _lrelu(e):
    return jnp.where(e >= 0.0, e, 0.2 * e)


BRA = 8
NCHUNKA = NCHUNK // BRA


def _sc_denom_kernel(src_hbm, dst_hbm, asad_hbm, den_hbm, p_hbm,
                     as_v, ad_v, s0, s1, d0, d1, p0, p1, den_l, acc_v, tmp_v,
                     tmp2_v, shr_den, se0, se1, pe0, pe1):
    cid = lax.axis_index("c")
    sid = lax.axis_index("s")
    row_base = (cid * 16 + sid) * NCHUNK

    pltpu.sync_copy(asad_hbm.at[0], as_v)
    pltpu.sync_copy(asad_hbm.at[1], ad_v)

    def zbody(i, _):
        den_l[pl.ds(i * 16, 16)] = jnp.zeros((16,), _f32)
        return 0
    lax.fori_loop(0, NP // 16, zbody, 0)

    sb, db, pb = (s0, s1), (d0, d1), (p0, p1)
    ssem, psem = (se0, se1), (pe0, pe1)
    stg = [None] * NCHUNKA
    pw = [None] * NCHUNKA

    def stage(ci):
        r0 = row_base + ci * BRA
        b = ci & 1
        return (pltpu.async_copy(src_hbm.at[pl.ds(r0, BRA)], sb[b], ssem[b]),
                pltpu.async_copy(dst_hbm.at[pl.ds(r0, BRA)], db[b], ssem[b]))

    stg[0] = stage(0)
    for ci in range(NCHUNKA):
        b = ci & 1
        if ci + 1 < NCHUNKA:
            stg[ci + 1] = stage(ci + 1)
        stg[ci][0].wait()
        stg[ci][1].wait()
        if ci >= 2:
            pw[ci - 2].wait()

        def lane(li, _):
            r = li >> 3
            sl = pl.ds((li & 7) * 16, 16)
            s = sb[b][r, sl]
            d = db[b][r, sl]
            e = _lrelu(plsc.load_gather(as_v, [s])
                       + plsc.load_gather(ad_v, [d]))
            p = jnp.exp(e)
            pb[b][r, sl] = p
            plsc.addupdate_scatter(den_l, [d], p)
            return 0
        lax.fori_loop(0, BRA * CH // 16, lane, 0)
        pw[ci] = pltpu.async_copy(
            pb[b], p_hbm.at[pl.ds(row_base + ci * BRA, BRA)], psem[b])
    pw[NCHUNKA - 2].wait()
    pw[NCHUNKA - 1].wait()

    pltpu.sync_copy(den_l, shr_den.at[sid])
    plsc.subcore_barrier()
    pltpu.sync_copy(shr_den.at[0, pl.ds(sid * SL, SL)], acc_v)

    tb = (tmp_v, tmp2_v)
    dsc = pltpu.async_copy(shr_den.at[1, pl.ds(sid * SL, SL)], tb[1], psem[1])
    for t in range(1, 16):
        b2 = t & 1
        nxt = None
        if t + 1 < 16:
            nxt = pltpu.async_copy(
                shr_den.at[t + 1, pl.ds(sid * SL, SL)], tb[(t + 1) & 1],
                psem[(t + 1) & 1])
        dsc.wait()

        def abody(i, _):
            sl = pl.ds(i * 16, 16)
            acc_v[sl] = acc_v[sl] + tb[b2][sl]
            return 0
        lax.fori_loop(0, SL // 16, abody, 0)
        dsc = nxt
    pltpu.sync_copy(acc_v, den_hbm.at[cid, pl.ds(sid * SL, SL)])


def _sc_denom(src2, dst2, asad):
    k = pl.kernel(
        _sc_denom_kernel,
        out_type=(jax.ShapeDtypeStruct((2, NP), _f32),
                  jax.ShapeDtypeStruct((EPAD // CH, CH), _f32)),
        mesh=_MESH,
        compiler_params=_SC_PARAMS,
        scratch_types=[
            pltpu.VMEM((NP,), _f32),
            pltpu.VMEM((NP,), _f32),
            pltpu.VMEM((BRA, CH), _i32),
            pltpu.VMEM((BRA, CH), _i32),
            pltpu.VMEM((BRA, CH), _i32),
            pltpu.VMEM((BRA, CH), _i32),
            pltpu.VMEM((BRA, CH), _f32),
            pltpu.VMEM((BRA, CH), _f32),
            pltpu.VMEM((NP,), _f32),
            pltpu.VMEM((SL,), _f32),
            pltpu.VMEM((SL,), _f32),
            pltpu.VMEM((SL,), _f32),
            pltpu.VMEM_SHARED((16, NP), _f32),
            pltpu.SemaphoreType.DMA,
            pltpu.SemaphoreType.DMA,
            pltpu.SemaphoreType.DMA,
            pltpu.SemaphoreType.DMA,
        ],
    )
    return k(src2, dst2, asad)


NBLK = 11
BCH = NCHUNK // NBLK


def _sc_agg_kernel(src_hbm, dst_hbm, p_hbm, h_hbm, agg0_hbm,
                   agg1_hbm, si0, si1, di0, di1, pv0, pv1, rows0, rows1,
                   out_shr, gs0, gs1, ss0, ss1, ts0, ts1):
    cid = lax.axis_index("c")
    sid = lax.axis_index("s")
    row_base = (cid * 16 + sid) * NCHUNK

    def zr(i, _):
        for j in range(8):
            rows0[i, pl.ds(j * 16, 16)] = jnp.zeros((16,), _f32)
        return 0
    lax.fori_loop(0, CH, zr, 0)
    for kk in range(SL // CH):
        pltpu.sync_copy(rows0, out_shr.at[pl.ds(sid * SL + kk * CH, CH)])

    plsc.subcore_barrier()

    rows = (rows0, rows1)
    sb, db, pb = (si0, si1), (di0, di1), (pv0, pv1)
    gsem = (gs0, gs1)
    ssem = (ss0, ss1)
    tsem = (ts0, ts1)

    def stage(g, b):
        r0 = row_base + g * BCH
        pltpu.async_copy(src_hbm.at[pl.ds(r0, BCH)], sb[b], tsem[b])
        pltpu.async_copy(dst_hbm.at[pl.ds(r0, BCH)], db[b], tsem[b])
        pltpu.async_copy(p_hbm.at[pl.ds(r0, BCH)], pb[b], tsem[b])

    def stage_wait(b):
        pltpu.make_async_copy(src_hbm.at[pl.ds(0, BCH)], sb[b], tsem[b]).wait()
        pltpu.make_async_copy(dst_hbm.at[pl.ds(0, BCH)], db[b], tsem[b]).wait()
        pltpu.make_async_copy(p_hbm.at[pl.ds(0, BCH)], pb[b], tsem[b]).wait()

    def run_block(g, bg, last):
        if not last:
            stage(g + 1, 1 - bg)
        stage_wait(bg)
        gd = [None] * BCH
        sd = [None] * BCH
        gd[0] = pltpu.async_copy(h_hbm.at[sb[bg].at[0]], rows[0], gsem[0])
        for k in range(BCH):
            b = k & 1
            if k + 1 < BCH:
                if k >= 1:
                    sd[k - 1].wait()
                gd[k + 1] = pltpu.async_copy(
                    h_hbm.at[sb[bg].at[k + 1]], rows[(k + 1) & 1],
                    gsem[(k + 1) & 1])
            gd[k].wait()
            kvec = jnp.full((16,), k, _i32)

            def sbody(i, _):
                i0 = i * 4
                wv = [plsc.load_gather(
                    pb[bg], [kvec, jnp.full((16,), u, _i32) + i0])
                    for u in range(4)]
                for u in range(4):
                    for j in range(8):
                        sl = pl.ds(j * 16, 16)
                        rows[b][i0 + u, sl] = rows[b][i0 + u, sl] * wv[u]
                return 0
            lax.fori_loop(0, CH // 4, sbody, 0)

            sd[k] = pltpu.async_copy(rows[b], out_shr.at[db[bg].at[k]],
                                     ssem[b], add=True)
        sd[BCH - 2].wait()
        sd[BCH - 1].wait()

    stage(0, 0)

    def pair(q, _):
        run_block(2 * q, 0, False)
        run_block(2 * q + 1, 1, False)
        return 0
    lax.fori_loop(0, NBLK // 2, pair, 0)
    run_block(NBLK - 1, 0, True)

    plsc.subcore_barrier()
    for kk in range(SL // CH):
        r0 = sid * SL + kk * CH

        @pl.when(cid == 0)
        def _():
            pltpu.sync_copy(out_shr.at[pl.ds(r0, CH)],
                            agg0_hbm.at[pl.ds(r0, CH)])

        @pl.when(cid == 1)
        def _():
            pltpu.sync_copy(out_shr.at[pl.ds(r0, CH)],
                            agg1_hbm.at[pl.ds(r0, CH)])


def _sc_agg(src2, dst2, p2, h):
    k = pl.kernel(
        _sc_agg_kernel,
        out_type=(jax.ShapeDtypeStruct((NP, D), _f32),
                  jax.ShapeDtypeStruct((NP, D), _f32)),
        mesh=_MESH,
        compiler_params=_SC_PARAMS,
        scratch_types=[
            pltpu.VMEM((BCH, CH), _i32),
            pltpu.VMEM((BCH, CH), _i32),
            pltpu.VMEM((BCH, CH), _i32),
            pltpu.VMEM((BCH, CH), _i32),
            pltpu.VMEM((BCH, CH), _f32),
            pltpu.VMEM((BCH, CH), _f32),
            pltpu.VMEM((CH, D), _f32),
            pltpu.VMEM((CH, D), _f32),
            pltpu.VMEM_SHARED((NP, D), _f32),
            pltpu.SemaphoreType.DMA,
            pltpu.SemaphoreType.DMA,
            pltpu.SemaphoreType.DMA,
            pltpu.SemaphoreType.DMA,
            pltpu.SemaphoreType.DMA,
            pltpu.SemaphoreType.DMA,
        ],
    )
    return k(src2, dst2, p2, h)


def kernel(x, edge_index, batch, W1, a_src1, a_dst1, b1, W2, a_src2, a_dst2,
           b2, W3, a_src3, a_dst3, b3, lin1_W, lin1_b, lin2_W, lin2_b):
    loop = jnp.arange(N, dtype=edge_index.dtype)
    src = jnp.concatenate([edge_index[0], loop])
    dst = jnp.concatenate([edge_index[1], loop])
    pad = N + jnp.arange(EPAD - src.shape[0], dtype=_i32) % (NP - N)
    src = jnp.concatenate([src.astype(_i32), pad])
    dst = jnp.concatenate([dst.astype(_i32), pad])

    x_pad = jnp.pad(x, ((0, NP - N), (0, 0)))
    batch2d = jnp.pad(batch.astype(_i32), (0, NP - N),
                      constant_values=G).reshape(1, NP)

    a21 = jnp.stack([a_src1, a_dst1], axis=1)
    a22 = jnp.stack([a_src2, a_dst2], axis=1)
    a23 = jnp.stack([a_src3, a_dst3], axis=1)

    src2 = src.reshape(EPAD // CH, CH)
    dst2 = dst.reshape(EPAD // CH, CH)

    h1, asad1 = _tc_layer1(x_pad, W1, a21)
    den1, p1 = _sc_denom(src2, dst2, asad1)
    agg1a, agg1b = _sc_agg(src2, dst2, p1, h1)

    h2, asad2 = _tc_layer23(agg1a, agg1b, den1, b1.reshape(1, D), W2, a22)
    den2, p2 = _sc_denom(src2, dst2, asad2)
    agg2a, agg2b = _sc_agg(src2, dst2, p2, h2)

    h3, asad3 = _tc_layer23(agg2a, agg2b, den2, b2.reshape(1, D), W3, a23)
    den3, p3 = _sc_denom(src2, dst2, asad3)
    agg3a, agg3b = _sc_agg(src2, dst2, p3, h3)

    return _tc_final(agg3a, agg3b, den3, b3.reshape(1, D), batch2d, lin1_W,
                     lin1_b.reshape(1, -1), lin2_W, lin2_b.reshape(1, -1))
